# baseline (speedup 1.0000x reference)
import functools

import jax
import jax.numpy as jnp
from jax import lax
from jax.experimental import pallas as pl
from jax.experimental.pallas import tpu as pltpu
from jax.experimental.pallas import tpu_sc as plsc

NC = 2
NS = 16
NW = NC * NS
LANES = 16

SEQS_PER_CHUNK = 2


def _make_sc_kernel(n_rows, seq_len, dim):
    chunk = SEQS_PER_CHUNK * seq_len
    assert n_rows % (NW * chunk) == 0
    per_w = n_rows // NW
    n_chunks = per_w // chunk
    n_vregs = dim // LANES

    mesh = plsc.VectorSubcoreMesh(core_axis_name="c", subcore_axis_name="s")

    @functools.partial(
        pl.kernel,
        out_type=jax.ShapeDtypeStruct((n_rows, dim), jnp.float32),
        mesh=mesh,
        scratch_types=[
            pltpu.VMEM((chunk,), jnp.int32),
            pltpu.VMEM((chunk, dim), jnp.float32),
            pltpu.VMEM((seq_len, dim), jnp.float32),
            pltpu.SemaphoreType.DMA,
        ],
        compiler_params=pltpu.CompilerParams(use_tc_tiling_on_sc=False),
    )
    def sc_kernel(idx_hbm, tok_hbm, pos_hbm, out_hbm, idx_v, rows_v, pos_v, sem):
        wid = lax.axis_index("s") * NC + lax.axis_index("c")
        w_base = wid * per_w

        pltpu.sync_copy(pos_hbm, pos_v)

        def chunk_body(g, carry):
            base = w_base + g * chunk
            pltpu.sync_copy(idx_hbm.at[pl.ds(base, chunk)], idx_v)
            pltpu.async_copy(tok_hbm.at[idx_v], rows_v, sem).wait()

            def add_body(s, c2):
                for c in range(n_vregs):
                    pv = pos_v[s, pl.ds(c * LANES, LANES)]
                    for q in range(SEQS_PER_CHUNK):
                        r = q * seq_len + s
                        rows_v[r, pl.ds(c * LANES, LANES)] = (
                            rows_v[r, pl.ds(c * LANES, LANES)] + pv
                        )
                return c2

            lax.fori_loop(0, seq_len, add_body, 0)
            pltpu.sync_copy(rows_v, out_hbm.at[pl.ds(base, chunk)])
            return carry

        lax.fori_loop(0, n_chunks, chunk_body, 0)

    return sc_kernel


def kernel(inputs, token_table, position_table):
    batch, seq_len = inputs.shape
    dim = token_table.shape[1]
    n_rows = batch * seq_len
    idx = inputs.reshape(n_rows).astype(jnp.int32)
    sc = _make_sc_kernel(n_rows, seq_len, dim)
    out = sc(idx, token_table, position_table)
    return out.reshape(batch, seq_len, dim)


# speedup vs baseline: 1.1273x; 1.1273x over previous
import functools

import jax
import jax.numpy as jnp
from jax import lax
from jax.experimental import pallas as pl
from jax.experimental.pallas import tpu as pltpu
from jax.experimental.pallas import tpu_sc as plsc

NC = 2
NS = 16
NW = NC * NS
LANES = 16
NBUF = 4


def _make_sc_kernel(n_rows, seq_len, dim):
    chunk = seq_len
    per_w = n_rows // NW
    n_chunks = per_w // chunk
    assert n_rows % (NW * chunk) == 0 and n_chunks % NBUF == 0
    n_vregs = dim // LANES

    mesh = plsc.VectorSubcoreMesh(core_axis_name="c", subcore_axis_name="s")

    @functools.partial(
        pl.kernel,
        out_type=jax.ShapeDtypeStruct((n_rows, dim), jnp.float32),
        mesh=mesh,
        scratch_types=[
            pltpu.VMEM((per_w,), jnp.int32),
            [pltpu.VMEM((chunk, dim), jnp.float32) for _ in range(NBUF)],
            pltpu.VMEM((seq_len, dim), jnp.float32),
            [pltpu.SemaphoreType.DMA for _ in range(NBUF)],
            [pltpu.SemaphoreType.DMA for _ in range(NBUF)],
        ],
        compiler_params=pltpu.CompilerParams(use_tc_tiling_on_sc=False),
    )
    def sc_kernel(idx_hbm, tok_hbm, pos_hbm, out_hbm,
                  idx_all, rows, pos_v, gsem, ssem):
        wid = lax.axis_index("s") * NC + lax.axis_index("c")
        w_base = wid * per_w

        pltpu.sync_copy(pos_hbm, pos_v)
        pltpu.sync_copy(idx_hbm.at[pl.ds(w_base, per_w)], idx_all)

        def idx_slice(g):
            return idx_all.at[pl.ds(pl.multiple_of(g * chunk, 8), chunk)]

        def out_slice(g):
            return out_hbm.at[pl.ds(pl.multiple_of(w_base + g * chunk, 8), chunk)]

        def gather_start(g, b):
            pltpu.async_copy(tok_hbm.at[idx_slice(g)], rows[b], gsem[b])

        def gather_wait(g, b):
            pltpu.make_async_copy(tok_hbm.at[idx_slice(g)], rows[b], gsem[b]).wait()

        def store_start(g, b):
            pltpu.async_copy(rows[b], out_slice(g), ssem[b])

        def store_wait(g, b):
            pltpu.make_async_copy(rows[b], out_slice(g), ssem[b]).wait()

        gather_start(0, 0)
        gather_start(1, 1)

        def loop_body(i, carry):
            for b in range(NBUF):
                g = NBUF * i + b
                gp = g + 2
                pb = (b + 2) % NBUF

                def prefetch():
                    def wait_prev_store():
                        store_wait(gp - NBUF, pb)
                    if b >= 2:
                        wait_prev_store()
                    else:
                        pl.when(i >= 1)(wait_prev_store)
                    gather_start(gp, pb)

                if b < 2:
                    prefetch()
                else:
                    pl.when(i < n_chunks // NBUF - 1)(prefetch)

                gather_wait(g, b)

                @plsc.parallel_loop(0, seq_len, unroll=8)
                def add_body(s):
                    for c in range(n_vregs):
                        rows[b][s, pl.ds(c * LANES, LANES)] = (
                            rows[b][s, pl.ds(c * LANES, LANES)]
                            + pos_v[s, pl.ds(c * LANES, LANES)]
                        )

                store_start(g, b)
            return carry

        lax.fori_loop(0, n_chunks // NBUF, loop_body, 0)

        for b in range(NBUF):
            store_wait(n_chunks - NBUF + b, b)

    return sc_kernel


def kernel(inputs, token_table, position_table):
    batch, seq_len = inputs.shape
    dim = token_table.shape[1]
    n_rows = batch * seq_len
    idx = inputs.reshape(n_rows).astype(jnp.int32)
    sc = _make_sc_kernel(n_rows, seq_len, dim)
    out = sc(idx, token_table, position_table)
    return out.reshape(batch, seq_len, dim)
